# baseline (speedup 1.0000x reference)
import functools
import math

import jax
import jax.numpy as jnp
import numpy as np
from jax.experimental import pallas as pl
from jax.experimental.pallas import tpu as pltpu

_B, _T, _K = 64, 64, 32
_IN_DIM, _HID, _ACT = 4096, 1024, 7
_NE, _NH, _NB = 8, 8, 2
_HD = _HID // _NH

_BF = jnp.bfloat16
_F32 = jnp.float32


def _np_rope_coeff(seq_len, rows, scale):
    inv_freq = 1.0 / (10000.0 ** (np.arange(0, _HD, 2, dtype=np.float64) / _HD))
    t = np.arange(seq_len, dtype=np.float64)
    freqs = t[:, None] * inv_freq[None, :]
    emb = np.concatenate([freqs, freqs], axis=-1)
    cos, sin = np.cos(emb), np.sin(emb)
    even = (np.arange(_HD) % 2) == 0
    a = np.where(even[None, :], -sin, 0.0)
    b = np.where(even[None, :], 0.0, sin)
    tile = lambda z: jnp.asarray(
        np.tile(z * scale, (rows // seq_len, _HID // _HD)).astype(np.float32))
    return tile(cos), tile(a), tile(b)


def _roll_l(z):
    return jnp.concatenate([z[:, 1:], z[:, :1]], axis=1)


def _roll_r(z):
    return jnp.concatenate([z[:, -1:], z[:, :-1]], axis=1)


def _rope(y, c, a, b):
    return y * c + _roll_l(y) * a + _roll_r(y) * b


def _mm_kernel(x_ref, w_ref, b_ref, o_ref):
    k = pl.program_id(1)

    @pl.when(k == 0)
    def _():
        o_ref[...] = jnp.zeros_like(o_ref)

    o_ref[...] += jnp.dot(x_ref[...], w_ref[...], preferred_element_type=_F32)

    @pl.when(k == pl.num_programs(1) - 1)
    def _():
        o_ref[...] += b_ref[...]


def _matmul_bias(x, w, b_row, mb, kb):
    m, kk = x.shape
    n = w.shape[1]
    return pl.pallas_call(
        _mm_kernel,
        grid=(m // mb, kk // kb),
        in_specs=[
            pl.BlockSpec((mb, kb), lambda i, k: (i, k)),
            pl.BlockSpec((kb, n), lambda i, k: (k, 0)),
            pl.BlockSpec((1, n), lambda i, k: (0, 0)),
        ],
        out_specs=pl.BlockSpec((mb, n), lambda i, k: (i, 0)),
        out_shape=jax.ShapeDtypeStruct((m, n), _F32),
        compiler_params=pltpu.CompilerParams(
            dimension_semantics=("parallel", "arbitrary")),
    )(x, w, b_row)


def _dual_rope_kernel(e_ref, x_ref, w1_ref, w2_ref, b1_ref, b2_ref,
                      c_ref, a_ref, b3_ref, o_ref, *, rope2):
    xb = x_ref[...].astype(_BF)
    c, a, b = c_ref[...], a_ref[...], b3_ref[...]
    acc1 = jnp.dot(xb, w1_ref[...].astype(_BF),
                   preferred_element_type=_F32) + b1_ref[...]
    o_ref[:, :_HID] = _rope(acc1, c, a, b).astype(_BF)
    acc2 = jnp.dot(xb, w2_ref[...].astype(_BF),
                   preferred_element_type=_F32) + b2_ref[...]
    if rope2:
        o_ref[:, _HID:] = _rope(acc2, c, a, b).astype(_BF)
    else:
        o_ref[:, _HID:] = acc2.astype(_BF)


def _qq_proj(e_arr, x, w_moe, b_moe, qtab, layer, mb):
    m = x.shape[0]
    wspec = lambda i: pl.BlockSpec(
        (None, None, None, _HID, _HID), lambda mi, e: (layer, i, e[0], 0, 0))
    bspec = lambda i: pl.BlockSpec(
        (None, 1, _HID), lambda mi, e: (layer * 8 * _NE + i * _NE + e[0], 0, 0))
    tspec = pl.BlockSpec((mb, _HID), lambda mi, e: (0, 0))
    return pl.pallas_call(
        functools.partial(_dual_rope_kernel, rope2=True),
        grid_spec=pltpu.PrefetchScalarGridSpec(
            num_scalar_prefetch=1,
            grid=(m // mb,),
            in_specs=[
                pl.BlockSpec((mb, _HID), lambda mi, e: (mi, 0)),
                wspec(0), wspec(3), bspec(0), bspec(3),
                tspec, tspec, tspec,
            ],
            out_specs=pl.BlockSpec((mb, 2 * _HID), lambda mi, e: (mi, 0)),
        ),
        out_shape=jax.ShapeDtypeStruct((m, 2 * _HID), _BF),
        compiler_params=pltpu.CompilerParams(
            dimension_semantics=("parallel",)),
    )(e_arr, x, w_moe, w_moe, b_moe, b_moe, *qtab)


def _kv_proj(e_arr, kv_in, w_moe, b_moe, ktab, mb):
    wspec = lambda which: pl.BlockSpec(
        (None, None, None, _HID, _HID),
        lambda mi, e: ((mi // 4) % 2, (mi // 8) * 3 + which, e[0], 0, 0))
    bspec = lambda which: pl.BlockSpec(
        (None, 1, _HID),
        lambda mi, e: (((mi // 4) % 2) * 8 * _NE
                       + ((mi // 8) * 3 + which) * _NE + e[0], 0, 0))
    tspec = pl.BlockSpec(
        (None, mb, _HID),
        lambda mi, e: ((mi // 8) * (1 + (mi // 4) % 2), 0, 0))
    return pl.pallas_call(
        functools.partial(_dual_rope_kernel, rope2=False),
        grid_spec=pltpu.PrefetchScalarGridSpec(
            num_scalar_prefetch=1,
            grid=(16,),
            in_specs=[
                pl.BlockSpec((mb, _HID),
                             lambda mi, e: ((mi // 8) * 4 + mi % 4, 0)),
                wspec(1), wspec(2), bspec(1), bspec(2),
                tspec, tspec, tspec,
            ],
            out_specs=pl.BlockSpec((mb, 2 * _HID), lambda mi, e: (mi, 0)),
        ),
        out_shape=jax.ShapeDtypeStruct((16 * mb, 2 * _HID), _BF),
        compiler_params=pltpu.CompilerParams(
            dimension_semantics=("arbitrary",)),
    )(e_arr, kv_in, w_moe, w_moe, b_moe, b_moe, *ktab)


def _oln_ffn_kernel(e_ref, a_ref, wo_ref, bo_ref, res_ref, g_ref, be_ref,
                    wf_ref, bf_ref, o_ref):
    y = jnp.dot(a_ref[...], wo_ref[...].astype(_BF),
                preferred_element_type=_F32) + bo_ref[...] + res_ref[...]
    mu = jnp.mean(y, axis=1, keepdims=True)
    d = y - mu
    var = jnp.mean(d * d, axis=1, keepdims=True)
    hn = (d * jax.lax.rsqrt(var + 1e-5) * g_ref[...] + be_ref[...])
    o_ref[...] = jnp.maximum(
        jnp.dot(hn.astype(_BF), wf_ref[...].astype(_BF),
                preferred_element_type=_F32) + bf_ref[...], 0.0)


def _oln_ffn_head_kernel(e_ref, a_ref, wo_ref, bo_ref, res_ref, g_ref,
                         be_ref, wf_ref, bf_ref, wout_ref, bout_ref, o_ref):
    y = jnp.dot(a_ref[...], wo_ref[...].astype(_BF),
                preferred_element_type=_F32) + bo_ref[...] + res_ref[...]
    mu = jnp.mean(y, axis=1, keepdims=True)
    d = y - mu
    var = jnp.mean(d * d, axis=1, keepdims=True)
    hn = (d * jax.lax.rsqrt(var + 1e-5) * g_ref[...] + be_ref[...])
    t = jnp.maximum(
        jnp.dot(hn.astype(_BF), wf_ref[...].astype(_BF),
                preferred_element_type=_F32) + bf_ref[...], 0.0)
    o_ref[...] = jnp.dot(t.astype(_BF), wout_ref[...],
                         preferred_element_type=_F32) + bout_ref[...]


def _expert_oln_ffn(e_arr, attn, w_moe, b_moe, res, norm_g, norm_b, layer,
                    mb, head=None):
    m = attn.shape[0]
    in_specs = [
        pl.BlockSpec((mb, _HID), lambda mi, e: (mi, 0)),
        pl.BlockSpec((None, None, None, _HID, _HID),
                     lambda mi, e: (layer, 6, e[0], 0, 0)),
        pl.BlockSpec((None, 1, _HID),
                     lambda mi, e: (layer * 8 * _NE + 6 * _NE + e[0], 0, 0)),
        pl.BlockSpec((mb, _HID), lambda mi, e: (mi, 0)),
        pl.BlockSpec((None, 1, _HID),
                     lambda mi, e: (layer * _NE + e[0], 0, 0)),
        pl.BlockSpec((None, 1, _HID),
                     lambda mi, e: (layer * _NE + e[0], 0, 0)),
        pl.BlockSpec((None, None, None, _HID, _HID),
                     lambda mi, e: (layer, 7, e[0], 0, 0)),
        pl.BlockSpec((None, 1, _HID),
                     lambda mi, e: (layer * 8 * _NE + 7 * _NE + e[0], 0, 0)),
    ]
    args = [e_arr, attn, w_moe, b_moe, res, norm_g, norm_b, w_moe, b_moe]
    if head is None:
        kern, n_out, out_dt = _oln_ffn_kernel, _HID, _F32
    else:
        kern, n_out, out_dt = _oln_ffn_head_kernel, 128, _F32
        in_specs += [pl.BlockSpec((_HID, 128), lambda mi, e: (0, 0)),
                     pl.BlockSpec((1, 128), lambda mi, e: (0, 0))]
        args += [head[0], head[1]]
    return pl.pallas_call(
        kern,
        grid_spec=pltpu.PrefetchScalarGridSpec(
            num_scalar_prefetch=1,
            grid=(m // mb,),
            in_specs=in_specs,
            out_specs=pl.BlockSpec((mb, n_out), lambda mi, e: (mi, 0)),
        ),
        out_shape=jax.ShapeDtypeStruct((m, n_out), out_dt),
        compiler_params=pltpu.CompilerParams(
            dimension_semantics=("parallel",)),
    )(*args)


def _attn_kernel(q_ref, kva_ref, kvt_ref, o_ref, nb):
    zero = jnp.zeros((_K, _HD), _BF)
    dn = (((1,), (1,)), ((), ()))
    for j in range(nb):
        rq = slice(j * _T, (j + 1) * _T)
        rk = slice(j * _K, (j + 1) * _K)
        ss = []
        for h in range(_NH):
            sl = slice(h * _HD, (h + 1) * _HD)
            slt = slice(_HID + h * _HD, _HID + (h + 1) * _HD)
            q2 = jnp.concatenate([q_ref[rq, sl], q_ref[rq, slt]], axis=1)
            k2 = jnp.concatenate(
                [jnp.concatenate([kva_ref[rk, sl], zero], axis=1),
                 jnp.concatenate([zero, kvt_ref[rk, sl]], axis=1)], axis=0)
            ss.append(jax.lax.dot_general(q2, k2, dn,
                                          preferred_element_type=_F32))
        for h in range(_NH):
            sl = slice(h * _HD, (h + 1) * _HD)
            slt = slice(_HID + h * _HD, _HID + (h + 1) * _HD)
            s = ss[h]
            mx = jnp.max(s, axis=1, keepdims=True)
            p = jnp.exp(s - mx)
            den = jnp.sum(p, axis=1, keepdims=True)
            w = (p * (1.0 / den)).astype(_BF)
            v2 = jnp.concatenate([kva_ref[rk, slt], kvt_ref[rk, slt]], axis=0)
            o_ref[rq, sl] = jnp.dot(w, v2,
                                    preferred_element_type=_F32).astype(_BF)


def _attention(qq, kv, layer, nb):
    grid = _B // nb
    kb = (nb * _K) // 128
    return pl.pallas_call(
        functools.partial(_attn_kernel, nb=nb),
        grid=(grid,),
        in_specs=[
            pl.BlockSpec((nb * _T, 2 * _HID), lambda i: (i, 0)),
            pl.BlockSpec((nb * _K, 2 * _HID),
                         lambda i: (layer * (16 // kb) + i, 0)),
            pl.BlockSpec((nb * _K, 2 * _HID),
                         lambda i: ((32 + layer * 16) // kb + i, 0)),
        ],
        out_specs=pl.BlockSpec((nb * _T, _HID), lambda i: (i, 0)),
        out_shape=jax.ShapeDtypeStruct((_B * _T, _HID), _BF),
        compiler_params=pltpu.CompilerParams(
            dimension_semantics=("parallel",)),
    )(qq, kv, kv)


def kernel(x, h_a, h_t, W_moe, b_moe, norm_g, norm_b, gate, W_in, b_in,
           W_out, b_out, expert_idx):
    e_arr = jnp.asarray(expert_idx, dtype=jnp.int32).reshape((1,))

    xf = x.reshape(_B * _T, _IN_DIM).astype(_BF)
    kv_in = jnp.concatenate([h_a.reshape(_B * _K, _HID),
                             h_t.reshape(_B * _K, _HID)], axis=0)
    bm3 = b_moe.reshape(_NB * 8 * _NE, 1, _HID)
    ng3 = norm_g.reshape(_NB * _NE, 1, _HID)
    nb3 = norm_b.reshape(_NB * _NE, 1, _HID)

    inv = 1.0 / math.sqrt(_HD)
    qtab = _np_rope_coeff(_T, 1024, inv)
    kc, ka, kb_ = _np_rope_coeff(_K, 512, 1.0)
    r0 = jax.nn.sigmoid(gate[0, e_arr[0]])
    r1 = jax.nn.sigmoid(gate[1, e_arr[0]])
    ktab = tuple(jnp.stack([z, z * r0, z * r1]) for z in (kc, ka, kb_))

    h = _matmul_bias(xf, W_in.astype(_BF), b_in.reshape(1, _HID),
                     mb=2048, kb=1024)
    kv = _kv_proj(e_arr, kv_in, W_moe, bm3, ktab, mb=512)

    w_out_p = jnp.zeros((_HID, 128), _F32).at[:, :_ACT].set(W_out).astype(_BF)
    b_out_p = jnp.zeros((1, 128), _F32).at[:, :_ACT].set(b_out.reshape(1, _ACT))

    out = None
    for layer in range(_NB):
        qq = _qq_proj(e_arr, h, W_moe, bm3, qtab, layer, mb=1024)
        attn = _attention(qq, kv, layer, nb=16)
        head = None if layer < _NB - 1 else (w_out_p, b_out_p)
        nxt = _expert_oln_ffn(e_arr, attn, W_moe, bm3, h, ng3, nb3, layer,
                              mb=1024, head=head)
        if head is None:
            h = nxt
        else:
            out = nxt
    return out[:, :_ACT].reshape(_B, _T, _ACT)
